# Initial kernel scaffold; baseline (speedup 1.0000x reference)
import jax
import jax.numpy as jnp
from jax.experimental import pallas as pl


def kernel(x, weights):
    raise NotImplementedError("write your pallas kernel here")


# speedup vs baseline: 226.1842x; 226.1842x over previous
import functools

import jax
import jax.numpy as jnp
from jax import lax
from jax.experimental import pallas as pl
from jax.experimental.pallas import tpu as pltpu
from jax.experimental.pallas import tpu_sc as plsc

N = 128
PN = N + 3
PFLAT = PN * PN
PFLAT_PAD = 17168
B = 4096
NC = 2
NS = 16
NW = NC * NS
BQ = B // NW
LANES = 16
STEPS = BQ // LANES
SCALE = (N - 1) / 2.0


def _spline_basis(t):
    a0 = t + 1.0
    b0 = ((-0.5 * a0 + 2.5) * a0 - 4.0) * a0 + 2.0
    b1 = (1.5 * t - 2.5) * t * t + 1.0
    s = 1.0 - t
    b2 = (1.5 * s - 2.5) * s * s + 1.0
    a3 = 2.0 - t
    b3 = ((-0.5 * a3 + 2.5) * a3 - 4.0) * a3 + 2.0
    return b0, b1, b2, b3


@functools.partial(
    pl.kernel,
    out_type=jax.ShapeDtypeStruct((B,), jnp.float32),
    mesh=plsc.VectorSubcoreMesh(
        core_axis_name="c", subcore_axis_name="s", num_cores=NC, num_subcores=NS
    ),
    compiler_params=pltpu.CompilerParams(needs_layout_passes=False),
    scratch_types=[
        pltpu.VMEM((PFLAT_PAD,), jnp.float32),
        pltpu.VMEM((BQ,), jnp.float32),
        pltpu.VMEM((BQ,), jnp.float32),
        pltpu.VMEM((BQ,), jnp.float32),
    ],
)
def _spline_sc(qx_hbm, qy_hbm, tab_hbm, out_hbm, tab_v, qx_v, qy_v, o_v):
    wid = lax.axis_index("s") * NC + lax.axis_index("c")
    base = wid * BQ
    pltpu.sync_copy(qx_hbm.at[pl.ds(base, BQ)], qx_v)
    pltpu.sync_copy(qy_hbm.at[pl.ds(base, BQ)], qy_v)
    pltpu.sync_copy(tab_hbm, tab_v)

    for i in range(STEPS):
        qx = qx_v[pl.ds(i * LANES, LANES)]
        qy = qy_v[pl.ds(i * LANES, LANES)]
        xn = (qx + 1.0) * SCALE
        yn = (qy + 1.0) * SCALE
        c = jnp.clip(xn.astype(jnp.int32), 0, N - 1)
        r = jnp.clip(yn.astype(jnp.int32), 0, N - 1)
        t = xn - c.astype(jnp.float32)
        u = yn - r.astype(jnp.float32)
        bx = _spline_basis(t)
        by = _spline_basis(u)
        origin = r * PN + c
        acc = jnp.zeros((LANES,), jnp.float32)
        for dr in range(4):
            row = jnp.zeros((LANES,), jnp.float32)
            for dc in range(4):
                w = plsc.load_gather(tab_v, [origin + (dr * PN + dc)])
                row = row + bx[dc] * w
            acc = acc + by[dr] * row
        o_v[pl.ds(i * LANES, LANES)] = acc

    pltpu.sync_copy(o_v, out_hbm.at[pl.ds(base, BQ)])


def kernel(x, weights):
    qx = x[:, 0]
    qy = x[:, 1]
    tab = jnp.zeros((PN, PN), jnp.float32).at[1 : N + 1, 1 : N + 1].set(
        weights[:, 0].reshape(N, N)
    )
    tab_flat = jnp.zeros((PFLAT_PAD,), jnp.float32).at[:PFLAT].set(
        tab.reshape(-1)
    )
    out = _spline_sc(qx, qy, tab_flat)
    return (out, x)
